# baseline (speedup 1.0000x reference)
import functools

import numpy as np

import jax
import jax.numpy as jnp
from jax.experimental import pallas as pl
from jax.experimental.pallas import tpu as pltpu


def _umerge_nchw_kernel(over_ref, up_ref, w_ref, b_ref, p_ref, out_ref, *,
                        c_over, c_out, h, w):
    x = up_ref[0]
    y = jax.lax.dot_general(w_ref[...], x, (((1,), (0,)), ((), ())),
                            preferred_element_type=jnp.float32)
    p = p_ref[...]
    cols = []
    for i in range(h):
        ch = jnp.concatenate(
            [y[d * c_out:(d + 1) * c_out, i * w:(i + 1) * w] for d in range(4)],
            axis=1)
        cols.append(jax.lax.dot_general(ch, p, (((1,), (0,)), ((), ())),
                                        preferred_element_type=jnp.float32))
    conv = jnp.concatenate(cols, axis=1)
    b_col = jnp.transpose(b_ref[...], (1, 0))
    conv = conv + b_col
    out_ref[0, :c_over, :] = over_ref[0]
    out_ref[0, c_over:, :] = conv.astype(out_ref.dtype)


def kernel(over_nchw, up_nchw, weight, bias):
    B, c_in, h, w = up_nchw.shape
    c_out = weight.shape[1]
    c_over, Ho, Wo = over_nchw.shape[1], over_nchw.shape[2], over_nchw.shape[3]
    H, W = 2 * h, 2 * w
    c_total = c_over + c_out
    out_dtype = up_nchw.dtype

    if Ho != H:
        bh = (Ho - H) // 2
        over_nchw = over_nchw[:, :, bh:bh + H, :]
    if Wo != W:
        bw = (Wo - W) // 2
        over_nchw = over_nchw[:, :, :, bw:bw + W]

    over2 = over_nchw.reshape(B, c_over, H * W)
    up2 = up_nchw.reshape(B, c_in, h * w)

    wmat = jnp.transpose(weight, (2, 3, 1, 0)).reshape(4 * c_out, c_in)
    b2 = bias.reshape(1, c_out).astype(jnp.float32)

    perm = np.zeros((4 * w, 4 * w), np.float32)
    r = np.arange(4 * w)
    di, dj, j = (r // w) // 2, (r // w) % 2, r % w
    perm[r, di * W + 2 * j + dj] = 1.0
    pmat = jnp.asarray(perm)

    itemsize = jnp.dtype(out_dtype).itemsize
    cost = pl.CostEstimate(
        flops=2 * B * h * w * c_in * 4 * c_out + 2 * B * h * c_out * 4 * w * 4 * w,
        transcendentals=0,
        bytes_accessed=(up2.size + over2.size + B * c_total * H * W) * itemsize,
    )

    body = functools.partial(_umerge_nchw_kernel, c_over=c_over, c_out=c_out,
                             h=h, w=w)
    out2 = pl.pallas_call(
        body,
        out_shape=jax.ShapeDtypeStruct((B, c_total, H * W), out_dtype),
        grid=(B,),
        in_specs=[
            pl.BlockSpec((1, c_over, H * W), lambda b: (b, 0, 0)),
            pl.BlockSpec((1, c_in, h * w), lambda b: (b, 0, 0)),
            pl.BlockSpec((4 * c_out, c_in), lambda b: (0, 0)),
            pl.BlockSpec((1, c_out), lambda b: (0, 0)),
            pl.BlockSpec((4 * w, 4 * w), lambda b: (0, 0)),
        ],
        out_specs=pl.BlockSpec((1, c_total, H * W), lambda b: (b, 0, 0)),
        compiler_params=pltpu.CompilerParams(
            dimension_semantics=("parallel",)),
        cost_estimate=cost,
    )(over2, up2, wmat, b2, pmat)

    return out2.reshape(B, c_total, H, W)
